# baseline (speedup 1.0000x reference)
import functools

import jax
import jax.numpy as jnp
from jax import lax
from jax.experimental import pallas as pl
from jax.experimental.pallas import tpu as pltpu
from jax.experimental.pallas import tpu_sc as plsc

NUM_FIELDS = 26
VOCAB = 100000
DENSE_DIM = 13
BATCH = 4096

NC = 2
NS = 16
L = 16
NW = NC * NS
B_PER_W = BATCH // NW
CHUNKS = B_PER_W // L


def _sc_body(sparse_ref, dense_ref, table_ref, params_ref, out_ref,
             idx_v, gath_v, dense_v, w_v, out_v, sem):
    wid = lax.axis_index("s") * NC + lax.axis_index("c")
    base = wid * B_PER_W

    pltpu.sync_copy(sparse_ref.at[:, pl.ds(base, B_PER_W)], idx_v)
    pltpu.sync_copy(dense_ref.at[:, pl.ds(base, B_PER_W)], dense_v)
    pltpu.sync_copy(params_ref, w_v)

    def off_body(c, carry):
        sl = pl.ds(c * L, L)
        for f in range(NUM_FIELDS):
            idx_v[f, sl] = idx_v[f, sl] + f * VOCAB
        return carry

    lax.fori_loop(0, CHUNKS, off_body, 0)

    copies = [
        pltpu.async_copy(table_ref.at[idx_v.at[f]], gath_v.at[f], sem)
        for f in range(NUM_FIELDS)
    ]
    for cp in copies:
        cp.wait()

    wvec = w_v[:]

    def sum_body(c, carry):
        sl = pl.ds(c * L, L)
        acc = gath_v[0, sl]
        for f in range(1, NUM_FIELDS):
            acc = acc + gath_v[f, sl]
        for d in range(DENSE_DIM):
            acc = acc + dense_v[d, sl] * wvec[d]
        acc = acc + (wvec[DENSE_DIM] + wvec[DENSE_DIM + 1])
        out_v[sl] = 1.0 / (1.0 + jnp.exp(-acc))
        return carry

    lax.fori_loop(0, CHUNKS, sum_body, 0)

    pltpu.sync_copy(out_v, out_ref.at[pl.ds(base, B_PER_W)])


@jax.jit
def _run(sparse_t, dense_t, table_flat, params):
    mesh = plsc.VectorSubcoreMesh(core_axis_name="c", subcore_axis_name="s")
    call = functools.partial(
        pl.kernel,
        mesh=mesh,
        out_type=jax.ShapeDtypeStruct((BATCH,), jnp.float32),
        scratch_types=[
            pltpu.VMEM((NUM_FIELDS, B_PER_W), jnp.int32),
            pltpu.VMEM((NUM_FIELDS, B_PER_W), jnp.float32),
            pltpu.VMEM((DENSE_DIM, B_PER_W), jnp.float32),
            pltpu.VMEM((L,), jnp.float32),
            pltpu.VMEM((B_PER_W,), jnp.float32),
            pltpu.SemaphoreType.DMA,
        ],
    )(_sc_body)
    return call(sparse_t, dense_t, table_flat, params)


def kernel(sparse_inputs, dense_inputs, tables, dense_W, dense_b, bias):
    sparse_t = jnp.transpose(sparse_inputs).astype(jnp.int32)
    dense_t = jnp.transpose(dense_inputs)
    table_flat = tables.reshape(NUM_FIELDS * VOCAB)
    params = jnp.concatenate([
        dense_W.reshape(DENSE_DIM),
        dense_b.reshape(1),
        bias.reshape(1),
        jnp.zeros((1,), jnp.float32),
    ])
    return _run(sparse_t, dense_t, table_flat, params)
